# baseline (speedup 1.0000x reference)
import functools

import jax
import jax.numpy as jnp
from jax import lax
from jax.experimental import pallas as pl
from jax.experimental.pallas import tpu as pltpu
from jax.experimental.pallas import tpu_sc as plsc

HIDDEN = 2048
NUM_CB = 32
CB_VOCAB = 2051
NC, NS, L = 2, 16, 16
NW = NC * NS
GP = 16
RING = 3
Q = 8
UNROLL = 2


def _emb_call(n_pos, audio_tok, text_ids, text_table, audio_table):
    ppw = n_pos // NW
    ngrp = ppw // GP
    mesh = plsc.VectorSubcoreMesh(core_axis_name="c", subcore_axis_name="s")

    UNITS = ([("a", p, q) for p in range(8) for q in range(4)]
             + [("t", 0, 0)]
             + [("a", p, q) for p in range(8, 16) for q in range(4)]
             + [("t", 1, 0)])
    NU = len(UNITS)

    @functools.partial(
        pl.kernel,
        out_type=jax.ShapeDtypeStruct((n_pos, HIDDEN), jnp.float32),
        mesh=mesh,
        scratch_types=[
            pltpu.VMEM((2, GP, NUM_CB), jnp.int32),
            pltpu.VMEM((ppw,), jnp.int32),
            pltpu.VMEM((RING * Q, HIDDEN), jnp.float32),
            pltpu.VMEM((GP, HIDDEN), jnp.float32),
            pltpu.VMEM((2, 2, L), jnp.int32),
            pltpu.VMEM((L,), jnp.int32),
            pltpu.SemaphoreType.DMA,
            pltpu.SemaphoreType.DMA,
            pltpu.SemaphoreType.DMA,
            pltpu.SemaphoreType.DMA,
            pltpu.SemaphoreType.DMA,
            pltpu.SemaphoreType.DMA,
        ],
    )
    def k(atok_hbm, tids_hbm, text_hbm, audio_hbm, out_hbm,
          atok_v, tids_v, bufs, out_v, idx_a, idx_t,
          sem_g0, sem_g1, sem_g2, sem_o0, sem_o1, sem_i):
        wid = lax.axis_index("s") * NC + lax.axis_index("c")
        lane = lax.iota(jnp.int32, 16)
        base_pos = wid * ppw
        pltpu.sync_copy(tids_hbm.at[pl.ds(base_pos, ppw)], tids_v)
        gsems = (sem_g0, sem_g1, sem_g2)
        osems = (sem_o0, sem_o1)

        def fire(u_raw, slot, g, pos0):
            if u_raw >= NU:
                u, du, pos0 = u_raw - NU, 1, pos0 + GP
            else:
                u, du = u_raw, 0
            kind, p, q = UNITS[u]
            bk = (g + du) & 1
            dst = bufs.at[pl.ds(slot * Q, Q)]
            if kind == "t":
                if p == 0:
                    idx_t[pl.ds(0, L)] = tids_v[pl.ds(pos0, GP)]
                src = text_hbm.at[idx_t.at[pl.ds(p * Q, Q)]]
            else:
                i = p & 1
                if q == 0:
                    v0 = atok_v[bk, p, pl.ds(0, L)]
                    v1 = atok_v[bk, p, pl.ds(L, L)]
                    idx_a[i, 0, pl.ds(0, L)] = jnp.where(
                        v0 == 0, 0, v0 + lane * CB_VOCAB)
                    idx_a[i, 1, pl.ds(0, L)] = jnp.where(
                        v1 == 0, 0, v1 + (lane + L) * CB_VOCAB)
                src = audio_hbm.at[idx_a.at[i, q >> 1, pl.ds((q & 1) * Q, Q)]]
            pltpu.async_copy(src, dst, gsems[slot])

        def wait_unit(slot):
            pltpu.make_async_copy(audio_hbm.at[pl.ds(0, Q)],
                                  bufs.at[pl.ds(slot * Q, Q)],
                                  gsems[slot]).wait()

        def acc(u, slot):
            kind, p, q = UNITS[u]
            b0 = slot * Q

            @plsc.parallel_loop(0, HIDDEN // L, unroll=UNROLL)
            def _(c):
                off = c * L
                if kind == "t":
                    for j in range(Q):
                        plsc.addupdate(out_v.at[p * Q + j, pl.ds(off, L)],
                                       bufs[b0 + j, pl.ds(off, L)])
                else:
                    s = bufs[b0, pl.ds(off, L)]
                    for j in range(1, Q):
                        s = s + bufs[b0 + j, pl.ds(off, L)]
                    if q == 0:
                        out_v[p, pl.ds(off, L)] = s
                    else:
                        plsc.addupdate(out_v.at[p, pl.ds(off, L)], s)

        def drain_store(g, half):
            dst = out_hbm.at[pl.ds(base_pos + g * GP + half * Q, Q)]
            pltpu.make_async_copy(
                out_v.at[pl.ds(half * Q, Q)], dst, osems[half]).wait()

        def prefetch_atok(g, pos0):
            pltpu.async_copy(
                atok_hbm.at[pl.ds(base_pos + pos0 + GP, GP)],
                atok_v.at[(g + 1) & 1], sem_i)

        def drain_atok(g, pos0):
            pltpu.make_async_copy(
                atok_hbm.at[pl.ds(base_pos + pos0 + GP, GP)],
                atok_v.at[(g + 1) & 1], sem_i).wait()

        pltpu.sync_copy(atok_hbm.at[pl.ds(base_pos, GP)], atok_v.at[0])
        for u in range(RING):
            fire(u, u % RING, 0, 0)

        def group_body(g, _):
            pos0 = g * GP

            @pl.when(g < ngrp - 1)
            def _():
                prefetch_atok(g, pos0)

            @pl.when(g > 0)
            def _():
                drain_store(g - 1, 0)
            for u in range(NU):
                slot = u % RING
                wait_unit(slot)
                if UNITS[u] == ("a", 8, 0):
                    @pl.when(g > 0)
                    def _():
                        drain_store(g - 1, 1)
                acc(u, slot)
                if UNITS[u] == ("t", 0, 0):
                    pltpu.async_copy(
                        out_v.at[pl.ds(0, Q)],
                        out_hbm.at[pl.ds(base_pos + pos0, Q)], sem_o0)
                uf = u + RING
                if uf < NU:
                    fire(uf, uf % RING, g, pos0)
                else:
                    if uf == NU:
                        @pl.when(g < ngrp - 1)
                        def _():
                            drain_atok(g, pos0)

                    @pl.when(g < ngrp - 1)
                    def _():
                        fire(uf, uf % RING, g, pos0)
            pltpu.async_copy(
                out_v.at[pl.ds(Q, Q)],
                out_hbm.at[pl.ds(base_pos + pos0 + Q, Q)], sem_o1)
            return 0

        lax.fori_loop(0, ngrp, group_body, 0)
        drain_store(ngrp - 1, 0)
        drain_store(ngrp - 1, 1)

    return k(audio_tok, text_ids, text_table, audio_table)


def kernel(input_ids, text_table, audio_table, audio_tokens_offsets):
    b, s, _ = input_ids.shape
    n_pos = b * s
    ids2 = input_ids.reshape(n_pos, NUM_CB + 1).astype(jnp.int32)
    audio_tok = ids2[:, :NUM_CB]
    text_ids = ids2[:, NUM_CB]
    out = _emb_call(n_pos, audio_tok, text_ids, text_table, audio_table)
    return out.reshape(b, s, HIDDEN)


# speedup vs baseline: 1.0225x; 1.0225x over previous
import functools

import jax
import jax.numpy as jnp
from jax import lax
from jax.experimental import pallas as pl
from jax.experimental.pallas import tpu as pltpu
from jax.experimental.pallas import tpu_sc as plsc

HIDDEN = 2048
NUM_CB = 32
CB_VOCAB = 2051
NC, NS, L = 2, 16, 16
NW = NC * NS
GP = 16
RING = 5
Q = 8
UNROLL = 2


def _emb_call(n_pos, audio_tok, text_ids, text_table, audio_table):
    ppw = n_pos // NW
    ngrp = ppw // GP
    mesh = plsc.VectorSubcoreMesh(core_axis_name="c", subcore_axis_name="s")

    UNITS = ([("a", p, q) for p in range(8) for q in range(4)]
             + [("t", 0, 0)]
             + [("a", p, q) for p in range(8, 16) for q in range(4)]
             + [("t", 1, 0)])
    NU = len(UNITS)

    @functools.partial(
        pl.kernel,
        out_type=jax.ShapeDtypeStruct((n_pos, HIDDEN), jnp.float32),
        mesh=mesh,
        scratch_types=[
            pltpu.VMEM((GP, NUM_CB), jnp.int32),
            pltpu.VMEM((ppw,), jnp.int32),
            pltpu.VMEM((RING * Q, HIDDEN), jnp.float32),
            pltpu.VMEM((GP, HIDDEN), jnp.float32),
            pltpu.VMEM((2, 2, L), jnp.int32),
            pltpu.VMEM((L,), jnp.int32),
            pltpu.SemaphoreType.DMA,
            pltpu.SemaphoreType.DMA,
            pltpu.SemaphoreType.DMA,
            pltpu.SemaphoreType.DMA,
            pltpu.SemaphoreType.DMA,
            pltpu.SemaphoreType.DMA,
            pltpu.SemaphoreType.DMA,
        ],
    )
    def k(atok_hbm, tids_hbm, text_hbm, audio_hbm, out_hbm,
          atok_v, tids_v, bufs, out_v, idx_a, idx_t,
          sem_g0, sem_g1, sem_g2, sem_g3, sem_g4, sem_o0, sem_o1):
        wid = lax.axis_index("s") * NC + lax.axis_index("c")
        lane = lax.iota(jnp.int32, 16)
        base_pos = wid * ppw
        pltpu.sync_copy(tids_hbm.at[pl.ds(base_pos, ppw)], tids_v)
        gsems = (sem_g0, sem_g1, sem_g2, sem_g3, sem_g4)
        osems = (sem_o0, sem_o1)

        def fire(pos0, u, slot):
            kind, p, q = UNITS[u]
            dst = bufs.at[pl.ds(slot * Q, Q)]
            if kind == "t":
                if p == 0:
                    idx_t[pl.ds(0, L)] = tids_v[pl.ds(pos0, GP)]
                src = text_hbm.at[idx_t.at[pl.ds(p * Q, Q)]]
            else:
                i = p & 1
                if q == 0:
                    v0 = atok_v[p, pl.ds(0, L)]
                    v1 = atok_v[p, pl.ds(L, L)]
                    idx_a[i, 0, pl.ds(0, L)] = jnp.where(
                        v0 == 0, 0, v0 + lane * CB_VOCAB)
                    idx_a[i, 1, pl.ds(0, L)] = jnp.where(
                        v1 == 0, 0, v1 + (lane + L) * CB_VOCAB)
                src = audio_hbm.at[idx_a.at[i, q >> 1, pl.ds((q & 1) * Q, Q)]]
            return pltpu.async_copy(src, dst, gsems[slot])

        def acc(u, slot):
            kind, p, q = UNITS[u]
            b0 = slot * Q

            @plsc.parallel_loop(0, HIDDEN // L, unroll=UNROLL)
            def _(c):
                off = c * L
                if kind == "t":
                    for j in range(Q):
                        plsc.addupdate(out_v.at[p * Q + j, pl.ds(off, L)],
                                       bufs[b0 + j, pl.ds(off, L)])
                else:
                    s = bufs[b0, pl.ds(off, L)]
                    for j in range(1, Q):
                        s = s + bufs[b0 + j, pl.ds(off, L)]
                    if q == 0:
                        out_v[p, pl.ds(off, L)] = s
                    else:
                        plsc.addupdate(out_v.at[p, pl.ds(off, L)], s)

        def drain_store(g, half):
            dst = out_hbm.at[pl.ds(base_pos + g * GP + half * Q, Q)]
            pltpu.make_async_copy(
                out_v.at[pl.ds(half * Q, Q)], dst, osems[half]).wait()

        def group_body(g, _):
            pos0 = g * GP
            pltpu.sync_copy(atok_hbm.at[pl.ds(base_pos + pos0, GP)], atok_v)

            @pl.when(g > 0)
            def _():
                drain_store(g - 1, 0)
            cps = {}
            for u in range(RING):
                cps[u] = fire(pos0, u, u % RING)
            for u in range(NU):
                slot = u % RING
                cps[u].wait()
                if UNITS[u] == ("a", 8, 0):
                    @pl.when(g > 0)
                    def _():
                        drain_store(g - 1, 1)
                acc(u, slot)
                if UNITS[u] == ("t", 0, 0):
                    pltpu.async_copy(
                        out_v.at[pl.ds(0, Q)],
                        out_hbm.at[pl.ds(base_pos + pos0, Q)], sem_o0)
                if u + RING < NU:
                    cps[u + RING] = fire(pos0, u + RING, slot)
            pltpu.async_copy(
                out_v.at[pl.ds(Q, Q)],
                out_hbm.at[pl.ds(base_pos + pos0 + Q, Q)], sem_o1)
            return 0

        lax.fori_loop(0, ngrp, group_body, 0)
        drain_store(ngrp - 1, 0)
        drain_store(ngrp - 1, 1)

    return k(audio_tok, text_ids, text_table, audio_table)


def kernel(input_ids, text_table, audio_table, audio_tokens_offsets):
    b, s, _ = input_ids.shape
    n_pos = b * s
    ids2 = input_ids.reshape(n_pos, NUM_CB + 1).astype(jnp.int32)
    audio_tok = ids2[:, :NUM_CB]
    text_ids = ids2[:, NUM_CB]
    out = _emb_call(n_pos, audio_tok, text_ids, text_table, audio_table)
    return out.reshape(b, s, HIDDEN)


# speedup vs baseline: 1.0409x; 1.0180x over previous
import functools

import jax
import jax.numpy as jnp
from jax import lax
from jax.experimental import pallas as pl
from jax.experimental.pallas import tpu as pltpu
from jax.experimental.pallas import tpu_sc as plsc

HIDDEN = 2048
HALF = HIDDEN // 2
NUM_CB = 32
CB_VOCAB = 2051
NC, NS, L = 2, 16, 16
NW = NC * NS
GP = 16
RING = 5
UNROLL = 2


def _emb_call(n_pos, audio_tok, text_ids, text_table, audio_table):
    ppw = n_pos // NW
    ngrp = 2 * (ppw // GP)
    nunit = ngrp * 2 * GP
    mesh = plsc.VectorSubcoreMesh(core_axis_name="c", subcore_axis_name="s")

    @functools.partial(
        pl.kernel,
        out_type=jax.ShapeDtypeStruct((n_pos, HIDDEN), jnp.float32),
        mesh=mesh,
        scratch_types=[
            pltpu.VMEM((2, GP, NUM_CB), jnp.int32),
            pltpu.VMEM((ppw,), jnp.int32),
            pltpu.VMEM((RING, L, HALF), jnp.float32),
            pltpu.VMEM((2, GP, HALF), jnp.float32),
            pltpu.SemaphoreType.DMA,
            pltpu.SemaphoreType.DMA,
            pltpu.SemaphoreType.DMA,
            pltpu.SemaphoreType.DMA,
            pltpu.SemaphoreType.DMA,
            pltpu.SemaphoreType.DMA,
            pltpu.SemaphoreType.DMA,
            pltpu.SemaphoreType.DMA,
        ],
    )
    def k(atok_hbm, tids_hbm, text_hbm, audio_hbm, out_hbm,
          atok_v, tids_v, bufs, out_v,
          sem_t, sem_g0, sem_g1, sem_g2, sem_g3, sem_g4, sem_o0, sem_o1):
        wid = lax.axis_index("s") * NC + lax.axis_index("c")
        lane = lax.iota(jnp.int32, 16)
        base_pos = wid * ppw
        pltpu.sync_copy(atok_hbm.at[pl.ds(base_pos, GP)], atok_v.at[0])
        pltpu.sync_copy(tids_hbm.at[pl.ds(base_pos, ppw)], tids_v)
        gsems = (sem_g0, sem_g1, sem_g2, sem_g3, sem_g4)
        osems = (sem_o0, sem_o1)

        def fire(u, i):
            g = u >> 5
            h = g & 1
            bk = (g >> 1) & 1
            fp = (u >> 1) & (GP - 1)
            sh = u & 1
            v = atok_v[bk, fp, pl.ds(sh * L, L)]
            ix = jnp.where(v == 0, 0, v + (lane + sh * L) * CB_VOCAB)
            return pltpu.async_copy(
                audio_hbm.at[ix, pl.ds(h * HALF, HALF)], bufs.at[i], gsems[i])

        def fire_text(g):
            o = g & 1
            h = g & 1
            tix = tids_v[pl.ds((g >> 1) * GP, GP)]
            return pltpu.async_copy(
                text_hbm.at[tix, pl.ds(h * HALF, HALF)], out_v.at[o], sem_t)

        def acc(u, i):
            o = (u >> 5) & 1
            row = (u >> 1) & (GP - 1)

            @plsc.parallel_loop(0, HALF // L, unroll=UNROLL)
            def _(c):
                off = c * L
                s = bufs[i, 0, pl.ds(off, L)]
                for j in range(1, L):
                    s = s + bufs[i, j, pl.ds(off, L)]
                plsc.addupdate(out_v.at[o, row, pl.ds(off, L)], s)

        def out_dst(g):
            h = g & 1
            return out_hbm.at[pl.ds(base_pos + (g >> 1) * GP, GP),
                              pl.ds(h * HALF, HALF)]

        def store(g, parity):
            return pltpu.async_copy(out_v.at[parity], out_dst(g),
                                    osems[parity])

        def drain_store(g, parity):
            pltpu.make_async_copy(out_v.at[parity], out_dst(g),
                                  osems[parity]).wait()

        fire_text(0).wait()
        for i in range(RING):
            fire(i, i)

        def unit_body(u, _):
            g = u >> 5

            @pl.when(jnp.logical_and(u % 32 == 0, u > 0))
            def _():
                @pl.when(jnp.logical_and(g >= 2, g % 2 == 0))
                def _():
                    drain_store(g - 2, 0)

                @pl.when(jnp.logical_and(g >= 2, g % 2 == 1))
                def _():
                    drain_store(g - 2, 1)

                @pl.when(jnp.logical_and(g % 2 == 1, g < ngrp - 1))
                def _():
                    pb1 = (g >> 1) + 1
                    pltpu.sync_copy(
                        atok_hbm.at[pl.ds(base_pos + pb1 * GP, GP)],
                        atok_v.at[pb1 & 1])
                fire_text(g).wait()

            for i in range(RING):
                @pl.when(u % RING == i)
                def _():
                    pltpu.make_async_copy(
                        audio_hbm.at[pl.ds(0, L), pl.ds(0, HALF)],
                        bufs.at[i], gsems[i]).wait()
                    acc(u, i)
                    @pl.when(u + RING < nunit)
                    def _():
                        fire(u + RING, i)

            @pl.when(jnp.logical_and(u % 32 == 31, g % 2 == 0))
            def _():
                store(g, 0)

            @pl.when(jnp.logical_and(u % 32 == 31, g % 2 == 1))
            def _():
                store(g, 1)
            return 0

        lax.fori_loop(0, nunit, unit_body, 0)
        drain_store(ngrp - 2, (ngrp - 2) & 1)
        drain_store(ngrp - 1, (ngrp - 1) & 1)

    return k(audio_tok, text_ids, text_table, audio_table)


def kernel(input_ids, text_table, audio_table, audio_tokens_offsets):
    b, s, _ = input_ids.shape
    n_pos = b * s
    ids2 = input_ids.reshape(n_pos, NUM_CB + 1).astype(jnp.int32)
    audio_tok = ids2[:, :NUM_CB]
    text_ids = ids2[:, NUM_CB]
    out = _emb_call(n_pos, audio_tok, text_ids, text_table, audio_table)
    return out.reshape(b, s, HIDDEN)


# speedup vs baseline: 1.0488x; 1.0077x over previous
import functools

import jax
import jax.numpy as jnp
from jax import lax
from jax.experimental import pallas as pl
from jax.experimental.pallas import tpu as pltpu
from jax.experimental.pallas import tpu_sc as plsc

HIDDEN = 2048
NCOL = 4
CW = HIDDEN // NCOL
NUM_CB = 32
CB_VOCAB = 2051
NC, NS, L = 2, 16, 16
NW = NC * NS
GP = 16
RING = 8
UNROLL = 2


def _emb_call(n_pos, audio_tok, text_ids, text_table, audio_table):
    ppw = n_pos // NW
    ngrp = NCOL * (ppw // GP)
    nunit = ngrp * 2 * GP
    mesh = plsc.VectorSubcoreMesh(core_axis_name="c", subcore_axis_name="s")

    @functools.partial(
        pl.kernel,
        out_type=jax.ShapeDtypeStruct((n_pos, HIDDEN), jnp.float32),
        mesh=mesh,
        scratch_types=[
            pltpu.VMEM((2, GP, NUM_CB), jnp.int32),
            pltpu.VMEM((ppw,), jnp.int32),
            pltpu.VMEM((RING, L, CW), jnp.float32),
            pltpu.VMEM((2, GP, CW), jnp.float32),
            pltpu.SemaphoreType.DMA,
            pltpu.SemaphoreType.DMA,
            pltpu.SemaphoreType.DMA,
            pltpu.SemaphoreType.DMA,
            pltpu.SemaphoreType.DMA,
            pltpu.SemaphoreType.DMA,
            pltpu.SemaphoreType.DMA,
            pltpu.SemaphoreType.DMA,
            pltpu.SemaphoreType.DMA,
            pltpu.SemaphoreType.DMA,
            pltpu.SemaphoreType.DMA,
        ],
    )
    def k(atok_hbm, tids_hbm, text_hbm, audio_hbm, out_hbm,
          atok_v, tids_v, bufs, out_v,
          sem_t, sem_g0, sem_g1, sem_g2, sem_g3, sem_g4, sem_g5, sem_g6,
          sem_g7, sem_o0, sem_o1):
        wid = lax.axis_index("s") * NC + lax.axis_index("c")
        lane = lax.iota(jnp.int32, 16)
        base_pos = wid * ppw
        pltpu.sync_copy(atok_hbm.at[pl.ds(base_pos, GP)], atok_v.at[0])
        pltpu.sync_copy(tids_hbm.at[pl.ds(base_pos, ppw)], tids_v)
        gsems = (sem_g0, sem_g1, sem_g2, sem_g3, sem_g4, sem_g5, sem_g6,
                 sem_g7)
        osems = (sem_o0, sem_o1)

        def fire(u, i):
            g = u >> 5
            col = g & (NCOL - 1)
            bk = (g >> 2) & 1
            fp = (u >> 1) & (GP - 1)
            sh = u & 1
            v = atok_v[bk, fp, pl.ds(sh * L, L)]
            ix = jnp.where(v == 0, 0, v + (lane + sh * L) * CB_VOCAB)
            return pltpu.async_copy(
                audio_hbm.at[ix, pl.ds(col * CW, CW)], bufs.at[i], gsems[i])

        def fire_text(g):
            o = g & 1
            col = g & (NCOL - 1)
            tix = tids_v[pl.ds((g >> 2) * GP, GP)]
            return pltpu.async_copy(
                text_hbm.at[tix, pl.ds(col * CW, CW)], out_v.at[o], sem_t)

        def acc(u, i):
            o = (u >> 5) & 1
            row = (u >> 1) & (GP - 1)

            @plsc.parallel_loop(0, CW // L, unroll=UNROLL)
            def _(c):
                off = c * L
                s = bufs[i, 0, pl.ds(off, L)]
                for j in range(1, L):
                    s = s + bufs[i, j, pl.ds(off, L)]
                plsc.addupdate(out_v.at[o, row, pl.ds(off, L)], s)

        def out_dst(g):
            col = g & (NCOL - 1)
            return out_hbm.at[pl.ds(base_pos + (g >> 2) * GP, GP),
                              pl.ds(col * CW, CW)]

        def store(g, parity):
            return pltpu.async_copy(out_v.at[parity], out_dst(g),
                                    osems[parity])

        def drain_store(g, parity):
            pltpu.make_async_copy(out_v.at[parity], out_dst(g),
                                  osems[parity]).wait()

        fire_text(0).wait()
        for i in range(RING):
            fire(i, i)

        def it_body(it, _):
            for kk in range(8):
                u = it * 8 + kk
                if kk == 0:
                    g = u >> 5

                    @pl.when(jnp.logical_and(it % 4 == 0, it > 0))
                    def _():
                        @pl.when(jnp.logical_and(g >= 2, g % 2 == 0))
                        def _():
                            drain_store(g - 2, 0)

                        @pl.when(jnp.logical_and(g >= 2, g % 2 == 1))
                        def _():
                            drain_store(g - 2, 1)

                        @pl.when(jnp.logical_and(g & (NCOL - 1) == NCOL - 1,
                                                 g < ngrp - 1))
                        def _():
                            blk1 = (g >> 2) + 1
                            pltpu.sync_copy(
                                atok_hbm.at[pl.ds(base_pos + blk1 * GP, GP)],
                                atok_v.at[blk1 & 1])
                        fire_text(g).wait()
                pltpu.make_async_copy(
                    audio_hbm.at[pl.ds(0, L), pl.ds(0, CW)],
                    bufs.at[kk], gsems[kk]).wait()
                acc(u, kk)

                @pl.when(u + RING < nunit)
                def _():
                    fire(u + RING, kk)
                if kk == 7:
                    g7 = u >> 5

                    @pl.when(jnp.logical_and(it % 4 == 3, g7 % 2 == 0))
                    def _():
                        store(g7, 0)

                    @pl.when(jnp.logical_and(it % 4 == 3, g7 % 2 == 1))
                    def _():
                        store(g7, 1)
            return 0

        lax.fori_loop(0, nunit // 8, it_body, 0)
        drain_store(ngrp - 2, (ngrp - 2) & 1)
        drain_store(ngrp - 1, (ngrp - 1) & 1)

    return k(audio_tok, text_ids, text_table, audio_table)


def kernel(input_ids, text_table, audio_table, audio_tokens_offsets):
    b, s, _ = input_ids.shape
    n_pos = b * s
    ids2 = input_ids.reshape(n_pos, NUM_CB + 1).astype(jnp.int32)
    audio_tok = ids2[:, :NUM_CB]
    text_ids = ids2[:, NUM_CB]
    out = _emb_call(n_pos, audio_tok, text_ids, text_table, audio_table)
    return out.reshape(b, s, HIDDEN)
